# baseline (speedup 1.0000x reference)
import functools

import jax
import jax.numpy as jnp
from jax import lax
from jax.experimental import pallas as pl
from jax.experimental.pallas import tpu as pltpu
from jax.experimental.pallas import tpu_sc as plsc

VOCAB = 1000000
EMB_DIM = 64
BATCH = 16384
HIST = 200

NC, NS = 2, 16
NW = NC * NS
B_TOTAL = BATCH * HIST
B_PER_W = B_TOTAL // NW
CHUNK = 512
N_CHUNKS = B_PER_W // CHUNK

_MESH = plsc.VectorSubcoreMesh(
    core_axis_name="c", subcore_axis_name="s", num_cores=NC, num_subcores=NS
)


@functools.partial(
    pl.kernel,
    out_type=jax.ShapeDtypeStruct((B_TOTAL, EMB_DIM), jnp.float32),
    mesh=_MESH,
    scratch_types=[
        pltpu.VMEM((CHUNK,), jnp.int32),
        pltpu.VMEM((CHUNK, EMB_DIM), jnp.float32),
        pltpu.SemaphoreType.DMA,
    ],
    compiler_params=pltpu.CompilerParams(use_tc_tiling_on_sc=False),
)
def _gather_kernel(idx_hbm, table_hbm, out_hbm, idx_v, rows_v, sem):
    wid = lax.axis_index("s") * NC + lax.axis_index("c")
    base = wid * B_PER_W

    def step(i, carry):
        off = base + i * CHUNK
        pltpu.sync_copy(idx_hbm.at[pl.ds(off, CHUNK)], idx_v)
        pltpu.async_copy(table_hbm.at[idx_v], rows_v, sem).wait()
        pltpu.sync_copy(rows_v, out_hbm.at[pl.ds(off, CHUNK)])
        return carry

    lax.fori_loop(0, N_CHUNKS, step, 0)


def kernel(x, table):
    xf = x.reshape(-1).astype(jnp.int32)
    out = _gather_kernel(xf, table)
    return out.reshape(x.shape + (table.shape[-1],))


# speedup vs baseline: 1.0697x; 1.0697x over previous
import functools

import jax
import jax.numpy as jnp
from jax import lax
from jax.experimental import pallas as pl
from jax.experimental.pallas import tpu as pltpu
from jax.experimental.pallas import tpu_sc as plsc

VOCAB = 1000000
EMB_DIM = 64
BATCH = 16384
HIST = 200

NC, NS = 2, 16
NW = NC * NS
B_TOTAL = BATCH * HIST
B_PER_W = B_TOTAL // NW
CHUNK = 256
K = 5
GROUP = K * CHUNK
NG = B_PER_W // GROUP

_MESH = plsc.VectorSubcoreMesh(
    core_axis_name="c", subcore_axis_name="s", num_cores=NC, num_subcores=NS
)


@functools.partial(
    pl.kernel,
    out_type=jax.ShapeDtypeStruct((B_TOTAL, EMB_DIM), jnp.float32),
    mesh=_MESH,
    scratch_types=(
        [
            pltpu.VMEM((2, GROUP), jnp.int32),
            pltpu.VMEM((K, CHUNK, EMB_DIM), jnp.float32),
        ]
        + [pltpu.SemaphoreType.DMA] * K
        + [pltpu.SemaphoreType.DMA] * K
        + [pltpu.SemaphoreType.DMA] * 2
    ),
    compiler_params=pltpu.CompilerParams(use_tc_tiling_on_sc=False),
)
def _gather_kernel(idx_hbm, table_hbm, out_hbm, idx_v, rows_v, *sems):
    gsem = sems[:K]
    wsem = sems[K : 2 * K]
    isem = sems[2 * K :]
    wid = lax.axis_index("s") * NC + lax.axis_index("c")
    base = wid * B_PER_W

    pltpu.sync_copy(idx_hbm.at[pl.ds(base, GROUP)], idx_v.at[0])

    def group_body(g, carry):
        parity = lax.rem(g, 2)

        @pl.when(g + 1 < NG)
        def _():
            pltpu.async_copy(
                idx_hbm.at[pl.ds(base + (g + 1) * GROUP, GROUP)],
                idx_v.at[1 - parity],
                isem[0],
            )

        for b in range(K):

            @pl.when(g > 0)
            def _(b=b):
                pltpu.make_async_copy(
                    rows_v.at[b],
                    out_hbm.at[pl.ds(base, CHUNK)],
                    wsem[b],
                ).wait()

            pltpu.async_copy(
                table_hbm.at[idx_v.at[parity, pl.ds(b * CHUNK, CHUNK)]],
                rows_v.at[b],
                gsem[b],
            )

        for b in range(K):
            pltpu.make_async_copy(
                table_hbm.at[idx_v.at[parity, pl.ds(b * CHUNK, CHUNK)]],
                rows_v.at[b],
                gsem[b],
            ).wait()
            pltpu.async_copy(
                rows_v.at[b],
                out_hbm.at[pl.ds(base + (g * K + b) * CHUNK, CHUNK)],
                wsem[b],
            )

        @pl.when(g + 1 < NG)
        def _():
            pltpu.make_async_copy(
                idx_hbm.at[pl.ds(base, GROUP)], idx_v.at[1 - parity], isem[0]
            ).wait()

        return carry

    lax.fori_loop(0, NG, group_body, 0)

    for b in range(K):
        pltpu.make_async_copy(
            rows_v.at[b], out_hbm.at[pl.ds(base, CHUNK)], wsem[b]
        ).wait()


def kernel(x, table):
    xf = x.reshape(-1).astype(jnp.int32)
    out = _gather_kernel(xf, table)
    return out.reshape(x.shape + (table.shape[-1],))
